# baseline (speedup 1.0000x reference)
import functools

import jax
import jax.numpy as jnp
from jax import lax
from jax.experimental import pallas as pl
from jax.experimental.pallas import tpu as pltpu
from jax.experimental.pallas import tpu_sc as plsc

_B, _T = 2, 2048
_N = _B * _T
_D = 1024
_E = 8
_TN = 1024

_NC, _NS, _L = 2, 16, 16
_NW = _NC * _NS
_CHUNK = (_N * _E) // _NW


def _gate_kernel(x_ref, wg_ref, g_ref):
    g_ref[...] = jnp.dot(x_ref[...], wg_ref[...],
                         preferred_element_type=jnp.float32)


def _take(v, idx):
    return lax.gather(
        v, idx[:, None],
        dimension_numbers=lax.GatherDimensionNumbers(
            offset_dims=(), collapsed_slice_dims=(0,), start_index_map=(0,)),
        slice_sizes=(1,),
        mode=lax.GatherScatterMode.PROMISE_IN_BOUNDS)


def _router_body(g_buf, sc_buf):
    lane = lax.broadcasted_iota(jnp.int32, (_L,), 0)
    gid = lane & 7

    def step(j, carry):
        g = g_buf[pl.ds(j * _L, _L)]

        m = g
        for off in (1, 2, 4):
            m = jnp.maximum(m, _take(m, lane ^ off))
        p = jnp.exp(g - m)
        s = p
        for off in (1, 2, 4):
            s = s + _take(s, lane ^ off)
        sm = p / s

        def group_argmax(v):
            bv, bi = v, gid
            for off in (1, 2, 4):
                ov = _take(bv, lane ^ off)
                oi = _take(bi, lane ^ off)
                better = (ov > bv) | ((ov == bv) & (oi < bi))
                bv = jnp.where(better, ov, bv)
                bi = jnp.where(better, oi, bi)
            return bi

        a1 = group_argmax(g)
        m1 = gid == a1
        g2 = jnp.where(m1, jnp.float32(-3e38), g)
        a2 = group_argmax(g2)
        keep = m1 | (gid == a2)

        sc_buf[pl.ds(j * _L, _L)] = jnp.where(keep, sm, jnp.float32(0.0))
        return carry

    lax.fori_loop(0, _CHUNK // _L, step, 0)


_sc_mesh = plsc.VectorSubcoreMesh(
    core_axis_name="c", subcore_axis_name="s", num_cores=_NC)


@functools.partial(
    pl.kernel,
    mesh=_sc_mesh,
    out_type=jax.ShapeDtypeStruct((_N * _E,), jnp.float32),
    scratch_types=[
        pltpu.VMEM((_CHUNK,), jnp.float32),
        pltpu.VMEM((_CHUNK,), jnp.float32),
    ],
)
def _router(g_hbm, sc_hbm, g_buf, sc_buf):
    wid = lax.axis_index("s") * _NC + lax.axis_index("c")
    base = wid * _CHUNK
    pltpu.sync_copy(g_hbm.at[pl.ds(base, _CHUNK)], g_buf)
    _router_body(g_buf, sc_buf)
    pltpu.sync_copy(sc_buf, sc_hbm.at[pl.ds(base, _CHUNK)])


def _expert_kernel(x_ref, sc_ref, we_ref, out_ref):
    x = x_ref[...]
    sc = sc_ref[...]
    acc = jnp.zeros(out_ref.shape, jnp.float32)
    for e in range(_E):
        s_e = sc[:, e][:, None]
        acc += s_e * jnp.dot(x, we_ref[:, e * _D:(e + 1) * _D],
                             preferred_element_type=jnp.float32)
    out_ref[...] = acc


@jax.jit
def kernel(x, W_gate, W_exp):
    xf = x.reshape(_N, _D)

    g = pl.pallas_call(
        _gate_kernel,
        grid=(_N // _TN,),
        in_specs=[
            pl.BlockSpec((_TN, _D), lambda i: (i, 0)),
            pl.BlockSpec((_D, _E), lambda i: (0, 0)),
        ],
        out_specs=pl.BlockSpec((_TN, _E), lambda i: (i, 0)),
        out_shape=jax.ShapeDtypeStruct((_N, _E), jnp.float32),
    )(xf, W_gate)

    sc = _router(g.reshape(_N * _E)).reshape(_N, _E)

    out = pl.pallas_call(
        _expert_kernel,
        grid=(_N // _TN,),
        in_specs=[
            pl.BlockSpec((_TN, _D), lambda i: (i, 0)),
            pl.BlockSpec((_TN, _E), lambda i: (i, 0)),
            pl.BlockSpec((_D, _E * _D), lambda i: (0, 0)),
        ],
        out_specs=pl.BlockSpec((_TN, _D), lambda i: (i, 0)),
        out_shape=jax.ShapeDtypeStruct((_N, _D), jnp.float32),
    )(xf, sc, W_exp)
    return out.reshape(_B, _T, _D)


# speedup vs baseline: 1.3324x; 1.3324x over previous
import jax
import jax.numpy as jnp
from jax.experimental import pallas as pl
from jax.experimental.pallas import tpu as pltpu

_B, _T = 2, 2048
_D = 1024
_E = 8
_TN = 1024


def _moe_kernel(x_ref, wg_ref, we_ref, out_ref):
    x = x_ref[...]

    g = jnp.dot(x, wg_ref[...], preferred_element_type=jnp.float32)
    sm = jax.nn.softmax(g, axis=-1)

    e_ids = jax.lax.broadcasted_iota(jnp.int32, g.shape, 1)
    a1 = jnp.argmax(g, axis=-1, keepdims=True)
    m1 = e_ids == a1
    g2 = jnp.where(m1, -jnp.inf, g)
    a2 = jnp.argmax(g2, axis=-1, keepdims=True)
    m2 = e_ids == a2
    sc = jnp.where(m1 | m2, sm, 0.0)

    acc = jnp.zeros(out_ref.shape, jnp.float32)
    for e in range(_E):
        s_e = sc[:, e][:, None]
        acc += s_e * jnp.dot(x, we_ref[:, e * _D:(e + 1) * _D],
                             preferred_element_type=jnp.float32)
    out_ref[...] = acc


@jax.jit
def kernel(x, W_gate, W_exp):
    n = _B * _T
    xf = x.reshape(n, _D)
    out = pl.pallas_call(
        _moe_kernel,
        grid=(n // _TN,),
        in_specs=[
            pl.BlockSpec((_TN, _D), lambda i: (i, 0)),
            pl.BlockSpec((_D, _E), lambda i: (0, 0)),
            pl.BlockSpec((_D, _E * _D), lambda i: (0, 0)),
        ],
        out_specs=pl.BlockSpec((_TN, _D), lambda i: (i, 0)),
        out_shape=jax.ShapeDtypeStruct((n, _D), jnp.float32),
        compiler_params=pltpu.CompilerParams(
            dimension_semantics=("parallel",)),
    )(xf, W_gate, W_exp)
    return out.reshape(_B, _T, _D)


# speedup vs baseline: 1.3496x; 1.0129x over previous
import jax
import jax.numpy as jnp
from jax.experimental import pallas as pl
from jax.experimental.pallas import tpu as pltpu

_B, _T = 2, 2048
_D = 1024
_E = 8
_TN = 1024


def _moe_kernel(x_ref, wg_ref, we_hbm, out_ref, w_vmem, sems):
    i = pl.program_id(0)

    def _w_copy(e):
        return pltpu.make_async_copy(
            we_hbm.at[:, pl.ds(e * _D, _D)],
            w_vmem.at[:, pl.ds(e * _D, _D)],
            sems.at[e])

    @pl.when(i == 0)
    def _():
        for e in range(_E):
            _w_copy(e).start()

    x = x_ref[...]

    g = jnp.dot(x, wg_ref[...], preferred_element_type=jnp.float32)
    sm = jax.nn.softmax(g, axis=-1)

    e_ids = jax.lax.broadcasted_iota(jnp.int32, g.shape, 1)
    a1 = jnp.argmax(g, axis=-1, keepdims=True)
    m1 = e_ids == a1
    g2 = jnp.where(m1, -jnp.inf, g)
    a2 = jnp.argmax(g2, axis=-1, keepdims=True)
    m2 = e_ids == a2
    sc = jnp.where(m1 | m2, sm, 0.0)

    acc = jnp.zeros(out_ref.shape, jnp.float32)
    for e in range(_E):
        @pl.when(i == 0)
        def _(e=e):
            _w_copy(e).wait()

        s_e = sc[:, e][:, None]
        acc += s_e * jnp.dot(x, w_vmem[:, e * _D:(e + 1) * _D],
                             preferred_element_type=jnp.float32)
    out_ref[...] = acc


@jax.jit
def kernel(x, W_gate, W_exp):
    n = _B * _T
    xf = x.reshape(n, _D)
    out = pl.pallas_call(
        _moe_kernel,
        grid=(n // _TN,),
        in_specs=[
            pl.BlockSpec((_TN, _D), lambda i: (i, 0)),
            pl.BlockSpec((_D, _E), lambda i: (0, 0)),
            pl.BlockSpec(memory_space=pl.ANY),
        ],
        out_specs=pl.BlockSpec((_TN, _D), lambda i: (i, 0)),
        out_shape=jax.ShapeDtypeStruct((n, _D), jnp.float32),
        scratch_shapes=[
            pltpu.VMEM((_D, _E * _D), jnp.float32),
            pltpu.SemaphoreType.DMA((_E,)),
        ],
    )(xf, W_gate, W_exp)
    return out.reshape(_B, _T, _D)
